# baseline (speedup 1.0000x reference)
import functools
import jax
import jax.numpy as jnp
from jax import lax
from jax.experimental import pallas as pl
from jax.experimental.pallas import tpu as pltpu
from jax.experimental.pallas import tpu_sc as plsc

_N = 65536
_D = 1024
_NW = 32
_G = 16
_ROWS_PER_W = _N // _NW
_GROUPS = _ROWS_PER_W // _G
_GW = _G * _D


def _sc_body(x_hbm, o_hbm, buf):
    wid = lax.axis_index("s") * 2 + lax.axis_index("c")
    word0 = wid * (_ROWS_PER_W * _D)
    row_off = lax.iota(jnp.int32, _G) * _D

    def group_body(g, _):
        base = word0 + g * _GW
        pltpu.sync_copy(x_hbm.at[pl.ds(base, _GW)], buf)

        def col_body(c, acc):
            idx = row_off + c
            v = plsc.load_gather(buf, [idx])
            acc = acc + v
            plsc.store_scatter(buf, [idx], acc)
            return acc

        lax.fori_loop(0, _D, col_body, jnp.zeros((_G,), jnp.float32))
        pltpu.sync_copy(buf, o_hbm.at[pl.ds(base, _GW)])
        return 0

    lax.fori_loop(0, _GROUPS, group_body, 0)


@functools.partial(
    pl.kernel,
    out_type=jax.ShapeDtypeStruct((_N * _D,), jnp.float32),
    mesh=plsc.VectorSubcoreMesh(
        core_axis_name="c", subcore_axis_name="s", num_cores=2, num_subcores=16
    ),
    scratch_types=[pltpu.VMEM((_GW,), jnp.float32)],
    compiler_params=pltpu.CompilerParams(needs_layout_passes=False),
)
def _sc_cumsum(x_hbm, o_hbm, buf):
    _sc_body(x_hbm, o_hbm, buf)


def kernel(x):
    n, d = x.shape
    return _sc_cumsum(x.reshape(-1)).reshape(n, d)
